# baseline (speedup 1.0000x reference)
import functools

import jax
import jax.numpy as jnp
from jax import lax
from jax.experimental import pallas as pl
from jax.experimental.pallas import tpu as pltpu
from jax.experimental.pallas import tpu_sc as plsc

_TB = 256
_N_EMBED = 8192
_DIM = 256
_N_TOKENS = 16384

_NC = 2
_NS = 16
_NW = _NC * _NS
_B_PER_W = _N_TOKENS // _NW
_CHUNK = 128
_N_CHUNKS = _B_PER_W // _CHUNK


_WINDOWS = ((0, 2736), (2736, 5472), (5472, 8192))


def _dist_argmin_body(x_ref, w_ref, x2_ref, w2_ref, idx_ref):
    mm2 = lax.dot_general(
        x_ref[...], w_ref[...],
        dimension_numbers=(((1,), (1,)), ((), ())),
        preferred_element_type=jnp.float32,
    )
    dist = (x2_ref[...] + w2_ref[...]) - mm2
    iota = lax.broadcasted_iota(jnp.int32, dist.shape, 1)
    inf = jnp.float32(jnp.inf)
    cv = jnp.full((dist.shape[0],), inf, jnp.float32)
    ci = jnp.zeros((dist.shape[0],), jnp.int32)
    for lo, hi in _WINDOWS:
        dwin = jnp.where((iota >= lo) & (iota < hi), dist, inf)
        li = jnp.argmin(dwin, axis=1).astype(jnp.int32)
        lv = jnp.min(dwin, axis=1)
        take = lv < cv
        ci = jnp.where(take, li, ci)
        cv = jnp.where(take, lv, cv).astype(jnp.bfloat16).astype(jnp.float32)
    idx_ref[...] = ci


def _dist_argmin(flatten, embed_weight, x2, w2):
    grid = (_N_TOKENS // _TB,)
    return pl.pallas_call(
        _dist_argmin_body,
        grid=grid,
        in_specs=[
            pl.BlockSpec((_TB, _DIM), lambda i: (i, 0)),
            pl.BlockSpec((_N_EMBED, _DIM), lambda i: (0, 0)),
            pl.BlockSpec((_TB, 1), lambda i: (i, 0)),
            pl.BlockSpec((1, _N_EMBED), lambda i: (0, 0)),
        ],
        out_specs=pl.BlockSpec((_TB,), lambda i: (i,)),
        out_shape=jax.ShapeDtypeStruct((_N_TOKENS,), jnp.int32),
    )(flatten, embed_weight, x2, w2)


@functools.partial(
    pl.kernel,
    mesh=plsc.VectorSubcoreMesh(core_axis_name="c", subcore_axis_name="s"),
    out_type=jax.ShapeDtypeStruct((_N_TOKENS, _DIM), jnp.float32),
    scratch_types=[
        pltpu.VMEM((_CHUNK,), jnp.int32),
        pltpu.VMEM((_CHUNK,), jnp.int32),
        pltpu.VMEM((_CHUNK, _DIM), jnp.float32),
        pltpu.VMEM((_CHUNK, _DIM), jnp.float32),
        pltpu.SemaphoreType.DMA,
        pltpu.SemaphoreType.DMA,
    ],
)
def _sc_gather(table_hbm, idx_hbm, out_hbm, idx_v0, idx_v1, rows0, rows1,
               sem0, sem1):
    wid = lax.axis_index("s") * _NC + lax.axis_index("c")
    base = wid * _B_PER_W
    idx_bufs = (idx_v0, idx_v1)
    row_bufs = (rows0, rows1)
    sems = (sem0, sem1)
    pltpu.sync_copy(idx_hbm.at[pl.ds(base, _CHUNK)], idx_v0)
    cps = [pltpu.async_copy(table_hbm.at[idx_v0], rows0, sem0)]
    for c in range(_N_CHUNKS):
        if c + 1 < _N_CHUNKS:
            b = (c + 1) % 2
            off_n = base + (c + 1) * _CHUNK
            pltpu.sync_copy(idx_hbm.at[pl.ds(off_n, _CHUNK)], idx_bufs[b])
            cps.append(pltpu.async_copy(table_hbm.at[idx_bufs[b]],
                                        row_bufs[b], sems[b]))
        cps[c].wait()
        pltpu.sync_copy(row_bufs[c % 2], out_hbm.at[pl.ds(base + c * _CHUNK, _CHUNK)])


def kernel(inputs, embed_weight):
    n_embed, dim = embed_weight.shape
    flatten = inputs.reshape(-1, dim)
    x2 = jnp.sum(flatten ** 2, axis=1, keepdims=True)
    w2 = jnp.sum(embed_weight ** 2, axis=1)[None, :]
    idx = _dist_argmin(flatten + flatten, embed_weight, x2, w2)
    quantize = _sc_gather(embed_weight, idx)
    return (quantize.reshape(inputs.shape), idx.reshape(inputs.shape[:-1]))


# speedup vs baseline: 1.0031x; 1.0031x over previous
import functools

import jax
import jax.numpy as jnp
from jax import lax
from jax.experimental import pallas as pl
from jax.experimental.pallas import tpu as pltpu
from jax.experimental.pallas import tpu_sc as plsc

_TB = 256
_N_EMBED = 8192
_DIM = 256
_N_TOKENS = 16384

_NC = 2
_NS = 16
_NW = _NC * _NS
_B_PER_W = _N_TOKENS // _NW
_CHUNK = 128
_N_CHUNKS = _B_PER_W // _CHUNK


_WINDOWS = ((0, 2736), (2736, 5472), (5472, 8192))


def _dist_argmin_body(x_ref, w_ref, x2_ref, w2_ref, idx_ref):
    mm2 = lax.dot_general(
        x_ref[...], w_ref[...],
        dimension_numbers=(((1,), (1,)), ((), ())),
        preferred_element_type=jnp.float32,
    )
    dist = (x2_ref[...] + w2_ref[...]) - mm2
    iota = lax.broadcasted_iota(jnp.int32, dist.shape, 1)
    inf = jnp.float32(jnp.inf)
    cv = jnp.full((dist.shape[0],), inf, jnp.float32)
    ci = jnp.zeros((dist.shape[0],), jnp.int32)
    for lo, hi in _WINDOWS:
        dwin = jnp.where((iota >= lo) & (iota < hi), dist, inf)
        li = jnp.argmin(dwin, axis=1).astype(jnp.int32)
        lv = jnp.min(dwin, axis=1)
        take = lv < cv
        ci = jnp.where(take, li, ci)
        cv = jnp.where(take, lv, cv).astype(jnp.bfloat16).astype(jnp.float32)
    idx_ref[...] = ci


def _dist_argmin(flatten, embed_weight, x2, w2):
    grid = (_N_TOKENS // _TB,)
    return pl.pallas_call(
        _dist_argmin_body,
        grid=grid,
        in_specs=[
            pl.BlockSpec((_TB, _DIM), lambda i: (i, 0)),
            pl.BlockSpec((_N_EMBED, _DIM), lambda i: (0, 0)),
            pl.BlockSpec((_TB, 1), lambda i: (i, 0)),
            pl.BlockSpec((1, _N_EMBED), lambda i: (0, 0)),
        ],
        out_specs=pl.BlockSpec((_TB,), lambda i: (i,)),
        out_shape=jax.ShapeDtypeStruct((_N_TOKENS,), jnp.int32),
    )(flatten, embed_weight, x2, w2)


@functools.partial(
    pl.kernel,
    mesh=plsc.VectorSubcoreMesh(core_axis_name="c", subcore_axis_name="s"),
    out_type=jax.ShapeDtypeStruct((_N_TOKENS, _DIM), jnp.float32),
    scratch_types=[
        pltpu.VMEM((_CHUNK,), jnp.int32),
        pltpu.VMEM((_CHUNK,), jnp.int32),
        pltpu.VMEM((_CHUNK, _DIM), jnp.float32),
        pltpu.VMEM((_CHUNK, _DIM), jnp.float32),
        pltpu.SemaphoreType.DMA,
        pltpu.SemaphoreType.DMA,
    ],
)
def _sc_gather(table_hbm, idx_hbm, out_hbm, idx_v0, idx_v1, rows0, rows1,
               sem0, sem1):
    wid = lax.axis_index("s") * _NC + lax.axis_index("c")
    base = wid * _B_PER_W
    idx_bufs = (idx_v0, idx_v1)
    row_bufs = (rows0, rows1)
    sems = (sem0, sem1)
    pltpu.sync_copy(idx_hbm.at[pl.ds(base, _CHUNK)], idx_v0)
    cps = [pltpu.async_copy(table_hbm.at[idx_v0], rows0, sem0)]
    for c in range(_N_CHUNKS):
        if c + 1 < _N_CHUNKS:
            b = (c + 1) % 2
            off_n = base + (c + 1) * _CHUNK
            pltpu.sync_copy(idx_hbm.at[pl.ds(off_n, _CHUNK)], idx_bufs[b])
            cps.append(pltpu.async_copy(table_hbm.at[idx_bufs[b]],
                                        row_bufs[b], sems[b]))
        cps[c].wait()
        pltpu.sync_copy(row_bufs[c % 2], out_hbm.at[pl.ds(base + c * _CHUNK, _CHUNK)])


def kernel(inputs, embed_weight):
    n_embed, dim = embed_weight.shape
    flatten = inputs.reshape(-1, dim)
    x2 = jnp.sum(flatten ** 2, axis=1, keepdims=True)
    w2 = jnp.sum(embed_weight ** 2, axis=1)[None, :]
    idx = _dist_argmin(flatten, embed_weight + embed_weight, x2, w2)
    quantize = _sc_gather(embed_weight, idx)
    return (quantize.reshape(inputs.shape), idx.reshape(inputs.shape[:-1]))


# speedup vs baseline: 1.0813x; 1.0780x over previous
import functools

import jax
import jax.numpy as jnp
from jax import lax
from jax.experimental import pallas as pl
from jax.experimental.pallas import tpu as pltpu
from jax.experimental.pallas import tpu_sc as plsc

_TB = 512
_N_EMBED = 8192
_DIM = 256
_N_TOKENS = 16384

_NC = 2
_NS = 16
_NW = _NC * _NS
_B_PER_W = _N_TOKENS // _NW
_CHUNK = 128
_N_CHUNKS = _B_PER_W // _CHUNK


_WINDOWS = ((0, 2736), (2736, 5472), (5472, 8192))


def _dist_argmin_body(x_ref, w_ref, x2_ref, w2_ref, idx_ref):
    mm2 = lax.dot_general(
        x_ref[...], w_ref[...],
        dimension_numbers=(((1,), (1,)), ((), ())),
        preferred_element_type=jnp.float32,
    )
    dist = (x2_ref[...] + w2_ref[...]) - mm2
    iota = lax.broadcasted_iota(jnp.int32, dist.shape, 1)
    inf = jnp.float32(jnp.inf)
    cv = jnp.full((dist.shape[0],), inf, jnp.float32)
    ci = jnp.zeros((dist.shape[0],), jnp.int32)
    for lo, hi in _WINDOWS:
        dwin = jnp.where((iota >= lo) & (iota < hi), dist, inf)
        li = jnp.argmin(dwin, axis=1).astype(jnp.int32)
        lv = jnp.min(dwin, axis=1)
        take = lv < cv
        ci = jnp.where(take, li, ci)
        cv = jnp.where(take, lv, cv).astype(jnp.bfloat16).astype(jnp.float32)
    idx_ref[...] = ci


def _dist_argmin(flatten, embed_weight, x2, w2):
    grid = (_N_TOKENS // _TB,)
    return pl.pallas_call(
        _dist_argmin_body,
        grid=grid,
        in_specs=[
            pl.BlockSpec((_TB, _DIM), lambda i: (i, 0)),
            pl.BlockSpec((_N_EMBED, _DIM), lambda i: (0, 0)),
            pl.BlockSpec((_TB, 1), lambda i: (i, 0)),
            pl.BlockSpec((1, _N_EMBED), lambda i: (0, 0)),
        ],
        out_specs=pl.BlockSpec((_TB,), lambda i: (i,)),
        out_shape=jax.ShapeDtypeStruct((_N_TOKENS,), jnp.int32),
    )(flatten, embed_weight, x2, w2)


@functools.partial(
    pl.kernel,
    mesh=plsc.VectorSubcoreMesh(core_axis_name="c", subcore_axis_name="s"),
    out_type=jax.ShapeDtypeStruct((_N_TOKENS, _DIM), jnp.float32),
    scratch_types=[
        pltpu.VMEM((_CHUNK,), jnp.int32),
        pltpu.VMEM((_CHUNK,), jnp.int32),
        pltpu.VMEM((_CHUNK, _DIM), jnp.float32),
        pltpu.VMEM((_CHUNK, _DIM), jnp.float32),
        pltpu.SemaphoreType.DMA,
        pltpu.SemaphoreType.DMA,
    ],
)
def _sc_gather(table_hbm, idx_hbm, out_hbm, idx_v0, idx_v1, rows0, rows1,
               sem0, sem1):
    wid = lax.axis_index("s") * _NC + lax.axis_index("c")
    base = wid * _B_PER_W
    idx_bufs = (idx_v0, idx_v1)
    row_bufs = (rows0, rows1)
    sems = (sem0, sem1)
    pltpu.sync_copy(idx_hbm.at[pl.ds(base, _CHUNK)], idx_v0)
    cps = [pltpu.async_copy(table_hbm.at[idx_v0], rows0, sem0)]
    for c in range(_N_CHUNKS):
        if c + 1 < _N_CHUNKS:
            b = (c + 1) % 2
            off_n = base + (c + 1) * _CHUNK
            pltpu.sync_copy(idx_hbm.at[pl.ds(off_n, _CHUNK)], idx_bufs[b])
            cps.append(pltpu.async_copy(table_hbm.at[idx_bufs[b]],
                                        row_bufs[b], sems[b]))
        cps[c].wait()
        pltpu.sync_copy(row_bufs[c % 2], out_hbm.at[pl.ds(base + c * _CHUNK, _CHUNK)])


def kernel(inputs, embed_weight):
    n_embed, dim = embed_weight.shape
    flatten = inputs.reshape(-1, dim)
    x2 = jnp.sum(flatten ** 2, axis=1, keepdims=True)
    w2 = jnp.sum(embed_weight ** 2, axis=1)[None, :]
    idx = _dist_argmin(flatten, embed_weight + embed_weight, x2, w2)
    quantize = _sc_gather(embed_weight, idx)
    return (quantize.reshape(inputs.shape), idx.reshape(inputs.shape[:-1]))
